# baseline (speedup 1.0000x reference)
import functools

import jax
import jax.numpy as jnp
from jax import lax
from jax.experimental import pallas as pl
from jax.experimental.pallas import tpu as pltpu
from jax.experimental.pallas import tpu_sc as plsc

H = 2048
I = 2048
E = 8
T = 2048
BT = 128
NB = 24
S = NB * BT
NF = 8
BF = I // NF
NW = 32
TPW = T // NW
RC = 16
F32 = jnp.float32
BF16 = jnp.bfloat16
I32 = jnp.int32


def _plan_body(lgp_ref, tri_ref, slot_ref, be_ref, nb_ref, score_ref):
    lg = lgp_ref[...]
    mx = jnp.max(lg, axis=1, keepdims=True)
    lane = lax.broadcasted_iota(I32, (T, 128), 1)
    eid = jnp.min(jnp.where(lg == mx, lane, 9999), axis=1, keepdims=True)
    score_ref[...] = jax.nn.sigmoid(mx)
    oh = (eid == lane).astype(BF16)
    prefix = jnp.dot(tri_ref[...], oh, preferred_element_type=F32)
    counts = prefix[T - 1:T, :]
    nbl = jnp.floor((counts + 127.0) * (1.0 / 128.0))
    u = lax.broadcasted_iota(I32, (128, 128), 0)
    v = lax.broadcasted_iota(I32, (128, 128), 1)
    uexc = (u < v).astype(BF16)
    base = jnp.dot(nbl.astype(BF16), uexc, preferred_element_type=F32)
    nb_ref[...] = jnp.sum(nbl, axis=1, keepdims=True).astype(I32)
    ohf = oh.astype(F32)
    base_tok = jnp.sum(ohf * base, axis=1, keepdims=True)
    rank = jnp.sum(ohf * prefix, axis=1, keepdims=True)
    slot_ref[...] = (base_tok * float(BT) + rank - 1.0).astype(I32)
    bio = lax.broadcasted_iota(I32, (32, 128), 0).astype(F32)
    blane = lax.broadcasted_iota(I32, (32, 128), 1).astype(F32)
    cond = (bio >= base) & (bio < base + nbl) & (counts > 0)
    be_match = jnp.sum(jnp.where(cond, blane, 0.0), axis=1, keepdims=True)
    matched = jnp.sum(cond.astype(F32), axis=1, keepdims=True) > 0
    emax = jnp.max(jnp.where(counts > 0,
                             lax.broadcasted_iota(I32, (1, 128), 1).astype(F32),
                             -1.0), axis=1, keepdims=True)
    be_ref[...] = jnp.where(matched, be_match, emax).astype(I32)


def _plan(lgp, tri):
    return pl.pallas_call(
        _plan_body,
        out_shape=(
            jax.ShapeDtypeStruct((T, 1), I32),
            jax.ShapeDtypeStruct((32, 1), I32),
            jax.ShapeDtypeStruct((1, 1), I32),
            jax.ShapeDtypeStruct((T, 1), F32),
        ),
    )(lgp, tri)


def _gateup_body(x_ref, score_ref, gw_ref, uw_ref, h_ref, xsc_ref, gwb, uwb):
    t = pl.program_id(0)

    @pl.when(t == 0)
    def _():
        gwb[...] = gw_ref[...].astype(BF16)
        uwb[...] = uw_ref[...].astype(BF16)

    xb = x_ref[...]
    _p = (xb * score_ref[...]).astype(BF16)
    xsc_ref[...] = pltpu.bitcast(_p.reshape(2 * BT, H // 2), I32)
    xb16 = xb.astype(BF16)
    g = jnp.dot(xb16, gwb[...], preferred_element_type=F32)
    u = jnp.dot(xb16, uwb[...], preferred_element_type=F32)
    h_ref[...] = (g * jax.nn.sigmoid(g) * u).astype(BF16)


def _gateup(x, score, gw, uw):
    return pl.pallas_call(
        _gateup_body,
        grid=(T // BT,),
        in_specs=[
            pl.BlockSpec((BT, H), lambda t: (t, 0)),
            pl.BlockSpec((BT, 1), lambda t: (t, 0)),
            pl.BlockSpec((H, I), lambda t: (0, 0)),
            pl.BlockSpec((H, I), lambda t: (0, 0)),
        ],
        out_specs=[
            pl.BlockSpec((BT, I), lambda t: (t, 0)),
            pl.BlockSpec((BT, H // 2), lambda t: (t, 0)),
        ],
        out_shape=(
            jax.ShapeDtypeStruct((T, I), BF16),
            jax.ShapeDtypeStruct((T, H // 2), I32),
        ),
        scratch_shapes=[pltpu.VMEM((H, I), BF16), pltpu.VMEM((H, I), BF16)],
    )(x, score, gw, uw)


def _dispatch_body(slot_hbm, xsc_hbm, xs_hbm, idx_v, rows_v, sem):
    wid = lax.axis_index("s") * 2 + lax.axis_index("c")
    base = wid * TPW
    for j in range(TPW // RC):
        b = base + j * RC
        pltpu.sync_copy(slot_hbm.at[pl.ds(b, RC)], idx_v)
        pltpu.sync_copy(xsc_hbm.at[pl.ds(b, RC)], rows_v)
        pltpu.async_copy(rows_v, xs_hbm.at[idx_v], sem).wait()


def _dispatch(slot, xsc32):
    mesh = plsc.VectorSubcoreMesh(core_axis_name="c", subcore_axis_name="s")
    return pl.kernel(
        _dispatch_body,
        out_type=jax.ShapeDtypeStruct((S, H // 2), I32),
        mesh=mesh,
        scratch_types=[
            pltpu.VMEM((RC,), I32),
            pltpu.VMEM((RC, H // 2), I32),
            pltpu.SemaphoreType.DMA,
        ],
    )(slot, xsc32)


NFA = 2
BFA = I // NFA
NFB = 4
BFB = I // NFB


def _gup_body(be_s, nb_s, xs_ref, gate_ref, up_ref, h_ref):
    b = pl.program_id(1)

    @pl.when(b < nb_s[0])
    def _():
        xb = pltpu.bitcast(xs_ref[...], BF16).reshape(BT, H)
        g = jnp.dot(xb, gate_ref[0].astype(BF16), preferred_element_type=F32)
        u = jnp.dot(xb, up_ref[0].astype(BF16), preferred_element_type=F32)
        h_ref[...] = (g * jax.nn.sigmoid(g) * u).astype(BF16)


def _gup(be, nbl, xs32, gate_up_proj):
    grid_spec = pltpu.PrefetchScalarGridSpec(
        num_scalar_prefetch=2,
        grid=(NFA, NB),
        in_specs=[
            pl.BlockSpec((BT, H // 2), lambda f, b, be, nb: (b, 0)),
            pl.BlockSpec((1, H, BFA), lambda f, b, be, nb: (be[b], 0, f)),
            pl.BlockSpec((1, H, BFA), lambda f, b, be, nb: (be[b], 0, NFA + f)),
        ],
        out_specs=pl.BlockSpec((BT, BFA), lambda f, b, be, nb: (b, f)),
    )
    return pl.pallas_call(
        _gup_body,
        grid_spec=grid_spec,
        out_shape=jax.ShapeDtypeStruct((S, I), BF16),
        compiler_params=pltpu.CompilerParams(
            dimension_semantics=("arbitrary", "arbitrary")),
    )(be, nbl, xs32, gate_up_proj, gate_up_proj)


def _down_body(be_s, nb_s, h_ref, down_ref, ys_ref, dwe):
    b = pl.program_id(0)
    changed = jnp.logical_or(b == 0, be_s[b] != be_s[jnp.maximum(b - 1, 0)])

    @pl.when(changed)
    def _():
        dwe[...] = down_ref[0].astype(BF16)

    @pl.when(b < nb_s[0])
    def _():
        part = jnp.dot(h_ref[...], dwe[...], preferred_element_type=F32)
        ys_ref[...] = pltpu.bitcast(part.astype(BF16).reshape(2 * BT, H // 2), I32)


def _down(be, nbl, h_all, down_proj):
    grid_spec = pltpu.PrefetchScalarGridSpec(
        num_scalar_prefetch=2,
        grid=(NB,),
        in_specs=[
            pl.BlockSpec((BT, I), lambda b, be, nb: (b, 0)),
            pl.BlockSpec((1, I, H), lambda b, be, nb: (be[b], 0, 0)),
        ],
        out_specs=pl.BlockSpec((BT, H // 2), lambda b, be, nb: (b, 0)),
        scratch_shapes=[pltpu.VMEM((I, H), BF16)],
    )
    return pl.pallas_call(
        _down_body,
        grid_spec=grid_spec,
        out_shape=jax.ShapeDtypeStruct((S, H // 2), I32),
        compiler_params=pltpu.CompilerParams(
            dimension_semantics=("arbitrary",)),
    )(be, nbl, h_all, down_proj)


def _combine_body(slot_hbm, ys_hbm, routed_hbm, idx_v, rows_v, sem):
    wid = lax.axis_index("s") * 2 + lax.axis_index("c")
    base = wid * TPW
    for j in range(TPW // RC):
        b = base + j * RC
        pltpu.sync_copy(slot_hbm.at[pl.ds(b, RC)], idx_v)
        pltpu.async_copy(ys_hbm.at[idx_v], rows_v, sem).wait()
        pltpu.sync_copy(rows_v, routed_hbm.at[pl.ds(b, RC)])


def _combine(slot, ys32):
    mesh = plsc.VectorSubcoreMesh(core_axis_name="c", subcore_axis_name="s")
    return pl.kernel(
        _combine_body,
        out_type=jax.ShapeDtypeStruct((T, H // 2), I32),
        mesh=mesh,
        scratch_types=[
            pltpu.VMEM((RC,), I32),
            pltpu.VMEM((RC, H // 2), I32),
            pltpu.SemaphoreType.DMA,
        ],
    )(slot, ys32)


def _shd_body(h_ref, dw_ref, sh_ref, dwb):
    t = pl.program_id(0)

    @pl.when(t == 0)
    def _():
        dwb[...] = dw_ref[...].astype(BF16)

    sh_ref[...] = jnp.dot(h_ref[...], dwb[...], preferred_element_type=F32)


def _shd(h, dw):
    return pl.pallas_call(
        _shd_body,
        grid=(T // BT,),
        in_specs=[
            pl.BlockSpec((BT, I), lambda t: (t, 0)),
            pl.BlockSpec((I, H), lambda t: (0, 0)),
        ],
        out_specs=pl.BlockSpec((BT, H), lambda t: (t, 0)),
        out_shape=jax.ShapeDtypeStruct((T, H), F32),
        scratch_shapes=[pltpu.VMEM((I, H), BF16)],
    )(h, dw)


def _add_body(sh_ref, routed_ref, out_ref):
    routed = pltpu.bitcast(routed_ref[...], BF16).reshape(BT, H)
    out_ref[...] = sh_ref[...] + routed.astype(F32)


def _add(sh, routed32):
    return pl.pallas_call(
        _add_body,
        grid=(T // BT,),
        in_specs=[
            pl.BlockSpec((BT, H), lambda t: (t, 0)),
            pl.BlockSpec((BT, H // 2), lambda t: (t, 0)),
        ],
        out_specs=pl.BlockSpec((BT, H), lambda t: (t, 0)),
        out_shape=jax.ShapeDtypeStruct((T, H), F32),
    )(sh, routed32)


def kernel(hidden_states, router_weight, gate_up_proj, down_proj,
           shared_gate_w, shared_up_w, shared_down_w):
    x = hidden_states.reshape(T, H)
    logits = x @ router_weight
    lgp = jnp.concatenate(
        [logits, jnp.full((T, 128 - E), -1e30, dtype=F32)], axis=1)
    r = lax.broadcasted_iota(I32, (T, T), 0)
    c = lax.broadcasted_iota(I32, (T, T), 1)
    tri = (r >= c).astype(BF16)

    slot2, be2, nb2, score = _plan(lgp, tri)
    slot = slot2.reshape(T)
    h, xsc = _gateup(x, score, shared_gate_w, shared_up_w)
    xs32 = _dispatch(slot, xsc)
    bea = be2.reshape(32)
    nba = nb2.reshape(1)
    h_all = _gup(bea, nba, xs32, gate_up_proj)
    ys = _down(bea, nba, h_all, down_proj)
    sh = _shd(h, shared_down_w)
    routed32 = _combine(slot, ys)
    out = _add(sh, routed32)
    return out, logits


# speedup vs baseline: 1.0171x; 1.0171x over previous
import functools

import jax
import jax.numpy as jnp
from jax import lax
from jax.experimental import pallas as pl
from jax.experimental.pallas import tpu as pltpu
from jax.experimental.pallas import tpu_sc as plsc

H = 2048
I = 2048
E = 8
T = 2048
BT = 128
NB = 24
S = NB * BT
NF = 8
BF = I // NF
NW = 32
TPW = T // NW
RC = 16
F32 = jnp.float32
BF16 = jnp.bfloat16
I32 = jnp.int32


def _plan_body(lgp_ref, tri_ref, slot_ref, be_ref, nb_ref, score_ref):
    lg = lgp_ref[...]
    mx = jnp.max(lg, axis=1, keepdims=True)
    lane = lax.broadcasted_iota(I32, (T, 128), 1)
    eid = jnp.min(jnp.where(lg == mx, lane, 9999), axis=1, keepdims=True)
    score_ref[...] = jax.nn.sigmoid(mx)
    oh = (eid == lane).astype(BF16)
    prefix = jnp.dot(tri_ref[...], oh, preferred_element_type=F32)
    counts = prefix[T - 1:T, :]
    nbl = jnp.floor((counts + 127.0) * (1.0 / 128.0))
    u = lax.broadcasted_iota(I32, (128, 128), 0)
    v = lax.broadcasted_iota(I32, (128, 128), 1)
    uexc = (u < v).astype(BF16)
    base = jnp.dot(nbl.astype(BF16), uexc, preferred_element_type=F32)
    nb_ref[...] = jnp.sum(nbl, axis=1, keepdims=True).astype(I32)
    ohf = oh.astype(F32)
    base_tok = jnp.sum(ohf * base, axis=1, keepdims=True)
    rank = jnp.sum(ohf * prefix, axis=1, keepdims=True)
    slot_ref[...] = (base_tok * float(BT) + rank - 1.0).astype(I32)
    bio = lax.broadcasted_iota(I32, (32, 128), 0).astype(F32)
    blane = lax.broadcasted_iota(I32, (32, 128), 1).astype(F32)
    cond = (bio >= base) & (bio < base + nbl) & (counts > 0)
    be_match = jnp.sum(jnp.where(cond, blane, 0.0), axis=1, keepdims=True)
    matched = jnp.sum(cond.astype(F32), axis=1, keepdims=True) > 0
    emax = jnp.max(jnp.where(counts > 0,
                             lax.broadcasted_iota(I32, (1, 128), 1).astype(F32),
                             -1.0), axis=1, keepdims=True)
    be_ref[...] = jnp.where(matched, be_match, emax).astype(I32)


def _plan(lgp, tri):
    return pl.pallas_call(
        _plan_body,
        out_shape=(
            jax.ShapeDtypeStruct((T, 1), I32),
            jax.ShapeDtypeStruct((32, 1), I32),
            jax.ShapeDtypeStruct((1, 1), I32),
            jax.ShapeDtypeStruct((T, 1), F32),
        ),
    )(lgp, tri)


def _gateup_body(x_ref, score_ref, gw_ref, uw_ref, h_ref, xsc_ref, gwb, uwb):
    t = pl.program_id(0)

    @pl.when(t == 0)
    def _():
        gwb[...] = gw_ref[...].astype(BF16)
        uwb[...] = uw_ref[...].astype(BF16)

    xb = x_ref[...]
    _p = (xb * score_ref[...]).astype(BF16)
    xsc_ref[...] = pltpu.bitcast(_p.reshape(2 * BT, H // 2), I32)
    xb16 = xb.astype(BF16)
    g = jnp.dot(xb16, gwb[...], preferred_element_type=F32)
    u = jnp.dot(xb16, uwb[...], preferred_element_type=F32)
    h_ref[...] = (g * jax.nn.sigmoid(g) * u).astype(BF16)


def _gateup(x, score, gw, uw):
    return pl.pallas_call(
        _gateup_body,
        grid=(T // BT,),
        in_specs=[
            pl.BlockSpec((BT, H), lambda t: (t, 0)),
            pl.BlockSpec((BT, 1), lambda t: (t, 0)),
            pl.BlockSpec((H, I), lambda t: (0, 0)),
            pl.BlockSpec((H, I), lambda t: (0, 0)),
        ],
        out_specs=[
            pl.BlockSpec((BT, I), lambda t: (t, 0)),
            pl.BlockSpec((BT, H // 2), lambda t: (t, 0)),
        ],
        out_shape=(
            jax.ShapeDtypeStruct((T, I), BF16),
            jax.ShapeDtypeStruct((T, H // 2), I32),
        ),
        scratch_shapes=[pltpu.VMEM((H, I), BF16), pltpu.VMEM((H, I), BF16)],
    )(x, score, gw, uw)


def _dispatch_body(slot_hbm, xsc_hbm, xs_hbm, idx_v, rows_v, sem):
    wid = lax.axis_index("s") * 2 + lax.axis_index("c")
    base = wid * TPW
    for j in range(TPW // RC):
        b = base + j * RC
        pltpu.sync_copy(slot_hbm.at[pl.ds(b, RC)], idx_v)
        pltpu.sync_copy(xsc_hbm.at[pl.ds(b, RC)], rows_v)
        pltpu.async_copy(rows_v, xs_hbm.at[idx_v], sem).wait()


def _dispatch(slot, xsc32):
    mesh = plsc.VectorSubcoreMesh(core_axis_name="c", subcore_axis_name="s")
    return pl.kernel(
        _dispatch_body,
        out_type=jax.ShapeDtypeStruct((S, H // 2), I32),
        mesh=mesh,
        scratch_types=[
            pltpu.VMEM((RC,), I32),
            pltpu.VMEM((RC, H // 2), I32),
            pltpu.SemaphoreType.DMA,
        ],
    )(slot, xsc32)


NFA = 2
BFA = I // NFA
NFB = 4
BFB = I // NFB


def _gup_body(be_s, nb_s, xs_ref, gate_ref, up_ref, h_ref):
    b = pl.program_id(1)

    @pl.when(b < nb_s[0])
    def _():
        xb = pltpu.bitcast(xs_ref[...], BF16).reshape(BT, H)
        g = jnp.dot(xb, gate_ref[0].astype(BF16), preferred_element_type=F32)
        u = jnp.dot(xb, up_ref[0].astype(BF16), preferred_element_type=F32)
        h_ref[...] = (g * jax.nn.sigmoid(g) * u).astype(BF16)


def _gup(be, nbl, xs32, gate_up_proj):
    grid_spec = pltpu.PrefetchScalarGridSpec(
        num_scalar_prefetch=2,
        grid=(NFA, NB),
        in_specs=[
            pl.BlockSpec((BT, H // 2), lambda f, b, be, nb: (b, 0)),
            pl.BlockSpec((1, H, BFA), lambda f, b, be, nb: (be[b], 0, f)),
            pl.BlockSpec((1, H, BFA), lambda f, b, be, nb: (be[b], 0, NFA + f)),
        ],
        out_specs=pl.BlockSpec((BT, BFA), lambda f, b, be, nb: (b, f)),
    )
    return pl.pallas_call(
        _gup_body,
        grid_spec=grid_spec,
        out_shape=jax.ShapeDtypeStruct((S, I), BF16),
        compiler_params=pltpu.CompilerParams(
            dimension_semantics=("arbitrary", "arbitrary")),
    )(be, nbl, xs32, gate_up_proj, gate_up_proj)


def _down_body(be_s, nb_s, h_ref, down_ref, ys_ref, dwe):
    b = pl.program_id(0)
    changed = jnp.logical_or(b == 0, be_s[b] != be_s[jnp.maximum(b - 1, 0)])

    @pl.when(changed)
    def _():
        dwe[...] = down_ref[0].astype(BF16)

    @pl.when(b < nb_s[0])
    def _():
        part = jnp.dot(h_ref[...], dwe[...], preferred_element_type=F32)
        ys_ref[...] = pltpu.bitcast(part.astype(BF16).reshape(2 * BT, H // 2), I32)


def _down(be, nbl, h_all, down_proj):
    grid_spec = pltpu.PrefetchScalarGridSpec(
        num_scalar_prefetch=2,
        grid=(NB,),
        in_specs=[
            pl.BlockSpec((BT, I), lambda b, be, nb: (b, 0)),
            pl.BlockSpec((1, I, H), lambda b, be, nb: (be[b], 0, 0)),
        ],
        out_specs=pl.BlockSpec((BT, H // 2), lambda b, be, nb: (b, 0)),
        scratch_shapes=[pltpu.VMEM((I, H), BF16)],
    )
    return pl.pallas_call(
        _down_body,
        grid_spec=grid_spec,
        out_shape=jax.ShapeDtypeStruct((S, H // 2), I32),
        compiler_params=pltpu.CompilerParams(
            dimension_semantics=("arbitrary",)),
    )(be, nbl, h_all, down_proj)


def _combine_body(slot_hbm, ys_hbm, routed_hbm, idx_v, rows_v, sem):
    wid = lax.axis_index("s") * 2 + lax.axis_index("c")
    base = wid * TPW
    for j in range(TPW // RC):
        b = base + j * RC
        pltpu.sync_copy(slot_hbm.at[pl.ds(b, RC)], idx_v)
        pltpu.async_copy(ys_hbm.at[idx_v], rows_v, sem).wait()
        pltpu.sync_copy(rows_v, routed_hbm.at[pl.ds(b, RC)])


def _combine(slot, ys32):
    mesh = plsc.VectorSubcoreMesh(core_axis_name="c", subcore_axis_name="s")
    return pl.kernel(
        _combine_body,
        out_type=jax.ShapeDtypeStruct((T, H // 2), I32),
        mesh=mesh,
        scratch_types=[
            pltpu.VMEM((RC,), I32),
            pltpu.VMEM((RC, H // 2), I32),
            pltpu.SemaphoreType.DMA,
        ],
    )(slot, ys32)


def _final_body(h_ref, routed_ref, dw_ref, out_ref, dwb):
    t = pl.program_id(0)

    @pl.when(t == 0)
    def _():
        dwb[...] = dw_ref[...].astype(BF16)

    s = jnp.dot(h_ref[...], dwb[...], preferred_element_type=F32)
    routed = pltpu.bitcast(routed_ref[...], BF16).reshape(BT, H)
    out_ref[...] = s + routed.astype(F32)


def _final(h, routed32, dw):
    return pl.pallas_call(
        _final_body,
        grid=(T // BT,),
        in_specs=[
            pl.BlockSpec((BT, I), lambda t: (t, 0)),
            pl.BlockSpec((BT, H // 2), lambda t: (t, 0)),
            pl.BlockSpec((I, H), lambda t: (0, 0)),
        ],
        out_specs=pl.BlockSpec((BT, H), lambda t: (t, 0)),
        out_shape=jax.ShapeDtypeStruct((T, H), F32),
        scratch_shapes=[pltpu.VMEM((I, H), BF16)],
    )(h, routed32, dw)


def kernel(hidden_states, router_weight, gate_up_proj, down_proj,
           shared_gate_w, shared_up_w, shared_down_w):
    x = hidden_states.reshape(T, H)
    logits = x @ router_weight
    lgp = jnp.concatenate(
        [logits, jnp.full((T, 128 - E), -1e30, dtype=F32)], axis=1)
    r = lax.broadcasted_iota(I32, (T, T), 0)
    c = lax.broadcasted_iota(I32, (T, T), 1)
    tri = (r >= c).astype(BF16)

    slot2, be2, nb2, score = _plan(lgp, tri)
    slot = slot2.reshape(T)
    h, xsc = _gateup(x, score, shared_gate_w, shared_up_w)
    xs32 = _dispatch(slot, xsc)
    bea = be2.reshape(32)
    nba = nb2.reshape(1)
    h_all = _gup(bea, nba, xs32, gate_up_proj)
    ys = _down(bea, nba, h_all, down_proj)
    routed32 = _combine(slot, ys)
    out = _final(h, routed32, shared_down_w)
    return out, logits
